# Initial kernel scaffold; baseline (speedup 1.0000x reference)
import jax
import jax.numpy as jnp
from jax.experimental import pallas as pl


def kernel(hidden_states, router_w, gate_w, up_w, down_w):
    raise NotImplementedError("write your pallas kernel here")


# speedup vs baseline: 3.3294x; 3.3294x over previous
import functools

import jax
import jax.numpy as jnp
from jax import lax
from jax.experimental import pallas as pl
from jax.experimental.pallas import tpu as pltpu
from jax.experimental.pallas import tpu_sc as plsc

HIDDEN = 1024
INTER = 2048
NE = 8
T = 4096

NC = 2
NS = 16
NW = NC * NS
TPW = T // NW
CHUNK = 64
NCHUNK = TPW // CHUNK

SUB = 256
ICCH = 512
NIC = INTER // ICCH


def _router_body(x_ref, rw_ref, pos_ref, w16_ref, offs_ref):
    x = x_ref[...]
    rw = rw_ref[...]
    logits = lax.dot_general(x, rw, (((1,), (1,)), ((), ())),
                             preferred_element_type=jnp.float32)

    v1 = jnp.max(logits, axis=1, keepdims=True)
    eq = (logits == v1).astype(jnp.float32)
    r8 = lax.broadcasted_iota(jnp.int32, (NE, NE), 0)
    c8 = lax.broadcasted_iota(jnp.int32, (NE, NE), 1)
    tri_incl = (r8 <= c8).astype(jnp.float32)
    cum = lax.dot_general(eq, tri_incl, (((1,), (0,)), ((), ())),
                          preferred_element_type=jnp.float32)
    one_hot = eq * (cum == 1.0).astype(jnp.float32)

    neg_inf = jnp.float32(-jnp.inf)
    v2 = jnp.max(jnp.where(one_hot > 0.0, neg_inf, logits), axis=1,
                 keepdims=True)
    w = 1.0 / (1.0 + jnp.exp(v2 - v1))

    counts = jnp.sum(one_hot, axis=0, keepdims=True)
    r16 = lax.broadcasted_iota(jnp.int32, (NE, 16), 0)
    c16 = lax.broadcasted_iota(jnp.int32, (NE, 16), 1)
    tri16 = (r16 < c16).astype(jnp.float32)
    counts_col = counts.reshape(NE, 1)
    offs16 = jnp.sum(counts_col * tri16, axis=0, keepdims=True)

    CH = 256
    rr = lax.broadcasted_iota(jnp.int32, (CH, CH), 0)
    cc = lax.broadcasted_iota(jnp.int32, (CH, CH), 1)
    ltri = (cc < rr).astype(jnp.float32)
    ranks = []
    carry = jnp.zeros((1, NE), jnp.float32)
    for k in range(T // CH):
        oh = lax.slice(one_hot, (k * CH, 0), ((k + 1) * CH, NE))
        rk = lax.dot_general(ltri, oh, (((1,), (0,)), ((), ())),
                             preferred_element_type=jnp.float32) + carry
        ranks.append(rk)
        carry = carry + jnp.sum(oh, axis=0, keepdims=True)
    rank = jnp.concatenate(ranks, axis=0)

    offs8 = lax.slice(offs16, (0, 0), (1, NE))
    pos_f = jnp.sum(one_hot * (rank + offs8), axis=1, keepdims=True)
    pos_ref[...] = pos_f.astype(jnp.int32)
    w16_ref[...] = jnp.broadcast_to(w, (T, 128))
    offs_ref[...] = offs16.astype(jnp.int32)


def _router_call(xf, router_w):
    return pl.pallas_call(
        _router_body,
        out_shape=[
            jax.ShapeDtypeStruct((T, 1), jnp.int32),
            jax.ShapeDtypeStruct((T, 128), jnp.float32),
            jax.ShapeDtypeStruct((1, 16), jnp.int32),
        ],
    )(xf, router_w)


def _dispatch_body(x_hbm, w16_hbm, pos_hbm, xs_hbm, ws_hbm,
                   idx_v, xrows_v, wrows_v):
    wid = lax.axis_index("s") * NC + lax.axis_index("c")
    base0 = wid * TPW
    for c in range(NCHUNK):
        base = base0 + c * CHUNK
        pltpu.sync_copy(pos_hbm.at[pl.ds(base, CHUNK)], idx_v)
        pltpu.sync_copy(x_hbm.at[pl.ds(base, CHUNK)], xrows_v)
        pltpu.sync_copy(xrows_v, xs_hbm.at[idx_v])
        pltpu.sync_copy(w16_hbm.at[pl.ds(base, CHUNK)], wrows_v)
        pltpu.sync_copy(wrows_v, ws_hbm.at[idx_v])


def _dispatch_call(xf, w16, pos1):
    mesh = plsc.VectorSubcoreMesh(core_axis_name="c", subcore_axis_name="s")
    f = pl.kernel(
        _dispatch_body,
        out_type=[
            jax.ShapeDtypeStruct((T, HIDDEN), jnp.float32),
            jax.ShapeDtypeStruct((T, 128), jnp.float32),
        ],
        mesh=mesh,
        scratch_types=[
            pltpu.VMEM((CHUNK,), jnp.int32),
            pltpu.VMEM((CHUNK, HIDDEN), jnp.float32),
            pltpu.VMEM((CHUNK, 128), jnp.float32),
        ],
    )
    return f(xf, w16, pos1)


def _ffn_body(offs_ref, xs_ref, ws_ref, gate_ref, up_ref, down_ref, out_ref):
    e = pl.program_id(0)
    ic = pl.program_id(1)
    start_e = offs_ref[e]
    end_e = offs_ref[e + 1]
    s0 = (start_e // 8) * 8
    n_sub = lax.div(end_e - s0 + SUB - 1, SUB)

    gate_b = gate_ref[0]
    up_b = up_ref[0]
    down_b = down_ref[0]
    is_first = ic == 0
    is_last = ic == NIC - 1

    def body(j, _):
        start0 = s0 + j * SUB
        start = pl.multiple_of(jnp.minimum(start0, T - SUB), 8)
        x_sub = xs_ref[pl.ds(start, SUB), :]
        g = lax.dot_general(x_sub, gate_b, (((1,), (1,)), ((), ())),
                            preferred_element_type=jnp.float32)
        u = lax.dot_general(x_sub, up_b, (((1,), (1,)), ((), ())),
                            preferred_element_type=jnp.float32)
        a = g * (1.0 / (1.0 + jnp.exp(-g))) * u
        y = lax.dot_general(a, down_b, (((1,), (1,)), ((), ())),
                            preferred_element_type=jnp.float32)
        rows = start + lax.broadcasted_iota(jnp.int32, (SUB, 1), 0)
        lo = jnp.maximum(start0, start_e)
        lim = jnp.minimum(start0 + SUB, end_e)
        mask = (rows >= lo) & (rows < lim)
        cur = out_ref[pl.ds(start, SUB), :]
        wrow = ws_ref[pl.ds(start, SUB), :][:, 0:1]
        acc = jnp.where(is_first, y, cur + y)
        val = jnp.where(is_last, acc * wrow, acc)
        out_ref[pl.ds(start, SUB), :] = jnp.where(mask, val, cur)
        return 0

    lax.fori_loop(0, n_sub, body, 0)


def _ffn_call(offs16, xs, ws, gate_w, up_w, down_w):
    return pl.pallas_call(
        _ffn_body,
        grid=(NE, NIC),
        in_specs=[
            pl.BlockSpec(memory_space=pltpu.SMEM),
            pl.BlockSpec((T, HIDDEN), lambda e, ic: (0, 0)),
            pl.BlockSpec((T, 128), lambda e, ic: (0, 0)),
            pl.BlockSpec((1, ICCH, HIDDEN), lambda e, ic: (e, ic, 0)),
            pl.BlockSpec((1, ICCH, HIDDEN), lambda e, ic: (e, ic, 0)),
            pl.BlockSpec((1, HIDDEN, ICCH), lambda e, ic: (e, 0, ic)),
        ],
        out_specs=pl.BlockSpec((T, HIDDEN), lambda e, ic: (0, 0)),
        out_shape=jax.ShapeDtypeStruct((T, HIDDEN), jnp.float32),
        compiler_params=pltpu.CompilerParams(
            dimension_semantics=("arbitrary", "arbitrary"),
            vmem_limit_bytes=60 * 1024 * 1024,
        ),
    )(offs16, xs, ws, gate_w, up_w, down_w)


def _combine_body(ys_hbm, pos_hbm, out_hbm, idx_v, rows_v):
    wid = lax.axis_index("s") * NC + lax.axis_index("c")
    base0 = wid * TPW
    for c in range(NCHUNK):
        base = base0 + c * CHUNK
        pltpu.sync_copy(pos_hbm.at[pl.ds(base, CHUNK)], idx_v)
        pltpu.sync_copy(ys_hbm.at[idx_v], rows_v)
        pltpu.sync_copy(rows_v, out_hbm.at[pl.ds(base, CHUNK)])


def _combine_call(ys, pos1):
    mesh = plsc.VectorSubcoreMesh(core_axis_name="c", subcore_axis_name="s")
    f = pl.kernel(
        _combine_body,
        out_type=jax.ShapeDtypeStruct((T, HIDDEN), jnp.float32),
        mesh=mesh,
        scratch_types=[
            pltpu.VMEM((CHUNK,), jnp.int32),
            pltpu.VMEM((CHUNK, HIDDEN), jnp.float32),
        ],
    )
    return f(ys, pos1)


def kernel(hidden_states, router_w, gate_w, up_w, down_w):
    bsz, seq, h = hidden_states.shape
    xf = hidden_states.reshape(T, h)
    pos, w16, offs = _router_call(xf, router_w)
    pos1 = pos.reshape(T)
    offs16 = offs.reshape(16)
    xs, ws = _dispatch_call(xf, w16, pos1)
    ys = _ffn_call(offs16, xs, ws, gate_w, up_w, down_w)
    out = _combine_call(ys, pos1)
    return out.reshape(bsz, seq, h)
